# Initial kernel scaffold; baseline (speedup 1.0000x reference)
import jax
import jax.numpy as jnp
from jax.experimental import pallas as pl


def kernel(X, params):
    raise NotImplementedError("write your pallas kernel here")


# speedup vs baseline: 126.5932x; 126.5932x over previous
import functools

import jax
import jax.numpy as jnp
from jax import lax
from jax.experimental import pallas as pl
from jax.experimental.pallas import tpu as pltpu
from jax.experimental.pallas import tpu_sc as plsc

_A = 8
_G = 256
_D = 16
_B = 1024
_NW = 32
_GPW = _G // _NW
_NCH = _B // 16


def _sc_body(xt_hbm, tt_hbm, out_hbm, x_v, t_v, o_v):
    wid = lax.axis_index("c") * 16 + lax.axis_index("s")

    izero = jnp.zeros((16,), jnp.int32)
    ione = jnp.ones((16,), jnp.int32)
    fzero = jnp.zeros((16,), jnp.float32)

    def group_body(gi, carry):
        g = wid * _GPW + gi
        pltpu.sync_copy(xt_hbm.at[g], x_v)
        pltpu.sync_copy(tt_hbm.at[g], t_v)

        def chunk_body(c, carry2):
            base = c * 16
            v = [x_v[pl.ds(k * _B + base, 16)] for k in range(_A)]

            rank = [izero] * _A
            for k in range(_A):
                for j in range(k):
                    le = jnp.where(v[j] <= v[k], ione, izero)
                    rank[k] = rank[k] + le
                    rank[j] = rank[j] + (ione - le)

            mask = jnp.full((16,), 255, jnp.int32)
            s_prev = fzero
            acc = [fzero] * _D
            for r in range(_A):
                rsplat = jnp.full((16,), r, jnp.int32)
                eqs = [rank[k] == rsplat for k in range(_A)]
                s_r = fzero
                pw = izero
                for k in range(_A):
                    s_r = s_r + jnp.where(eqs[k], v[k], fzero)
                    pw = pw + jnp.where(eqs[k], jnp.full((16,), 1 << k, jnp.int32), izero)
                coef = s_r - s_prev
                for d in range(_D):
                    idx = mask + jnp.full((16,), d * 256, jnp.int32)
                    t = plsc.load_gather(t_v, [idx])
                    acc[d] = acc[d] + coef * t
                mask = mask - pw
                s_prev = s_r

            for d in range(_D):
                o_v[pl.ds(d * _B + base, 16)] = acc[d]
            return carry2

        lax.fori_loop(0, _NCH, chunk_body, 0)
        pltpu.sync_copy(o_v, out_hbm.at[g])
        return carry

    lax.fori_loop(0, _GPW, group_body, 0)


_sc_call = pl.kernel(
    _sc_body,
    out_type=jax.ShapeDtypeStruct((_G, _D * _B), jnp.float32),
    mesh=plsc.VectorSubcoreMesh(core_axis_name="c", subcore_axis_name="s"),
    compiler_params=pltpu.CompilerParams(needs_layout_passes=False),
    scratch_types=[
        pltpu.VMEM((_A * _B,), jnp.float32),
        pltpu.VMEM((_D * 256,), jnp.float32),
        pltpu.VMEM((_D * _B,), jnp.float32),
    ],
)


def kernel(X, params):
    xt = X.reshape(_B, _G, _A).transpose(1, 2, 0).reshape(_G, _A * _B)
    tt = params.transpose(0, 2, 1).reshape(_G, _D * 256)
    out = _sc_call(xt, tt)
    return out.reshape(_G, _D, _B).transpose(2, 0, 1).reshape(_B, _G * _D)


# speedup vs baseline: 140.1548x; 1.1071x over previous
import jax
import jax.numpy as jnp
from jax import lax
from jax.experimental import pallas as pl
from jax.experimental.pallas import tpu as pltpu
from jax.experimental.pallas import tpu_sc as plsc

_A = 8
_G = 256
_D = 16
_B = 1024
_NW = 32
_GPW = _G // _NW
_NCH = _B // 16
_TSZ = 1 << _A

_NET = [(0, 1), (2, 3), (4, 5), (6, 7),
        (0, 2), (1, 3), (4, 6), (5, 7),
        (1, 2), (5, 6), (0, 4), (3, 7),
        (1, 5), (2, 6),
        (1, 4), (3, 6),
        (2, 4), (3, 5),
        (3, 4)]


def _sc_body(xt_hbm, tt_hbm, out_hbm, x_v, t_v, o_v):
    wid = lax.axis_index("c") * 16 + lax.axis_index("s")

    pconst = [jnp.full((16,), 1 << k, jnp.int32) for k in range(_A)]
    mask0 = jnp.full((16,), 255, jnp.int32)

    def group_body(gi, carry):
        g = wid * _GPW + gi
        pltpu.sync_copy(xt_hbm.at[g], x_v)
        pltpu.sync_copy(tt_hbm.at[g], t_v)

        def chunk_body(c, carry2):
            base = c * 16
            v = [x_v[pl.ds(k * _B + base, 16)] for k in range(_A)]
            p = list(pconst)

            for a, b in _NET:
                le = v[a] <= v[b]
                va = jnp.minimum(v[a], v[b])
                vb = jnp.maximum(v[a], v[b])
                pa = jnp.where(le, p[a], p[b])
                pb = jnp.where(le, p[b], p[a])
                v[a], v[b], p[a], p[b] = va, vb, pa, pb

            mask = mask0
            acc = [None] * _D
            sprev = None
            for r in range(_A):
                coef = v[r] if r == 0 else v[r] - sprev
                for d in range(_D):
                    t = plsc.load_gather(
                        t_v.at[pl.ds(d * _TSZ, _TSZ)], [mask])
                    acc[d] = coef * t if r == 0 else acc[d] + coef * t
                if r < _A - 1:
                    mask = mask - p[r]
                sprev = v[r]

            for d in range(_D):
                o_v[pl.ds(d * _B + base, 16)] = acc[d]
            return carry2

        lax.fori_loop(0, _NCH, chunk_body, 0)
        pltpu.sync_copy(o_v, out_hbm.at[g])
        return carry

    lax.fori_loop(0, _GPW, group_body, 0)


_sc_call = pl.kernel(
    _sc_body,
    out_type=jax.ShapeDtypeStruct((_G, _D * _B), jnp.float32),
    mesh=plsc.VectorSubcoreMesh(core_axis_name="c", subcore_axis_name="s"),
    compiler_params=pltpu.CompilerParams(needs_layout_passes=False),
    scratch_types=[
        pltpu.VMEM((_A * _B,), jnp.float32),
        pltpu.VMEM((_D * _TSZ,), jnp.float32),
        pltpu.VMEM((_D * _B,), jnp.float32),
    ],
)


def kernel(X, params):
    xt = X.reshape(_B, _G, _A).transpose(1, 2, 0).reshape(_G, _A * _B)
    tt = params.transpose(0, 2, 1).reshape(_G, _D * _TSZ)
    out = _sc_call(xt, tt)
    return out.reshape(_G, _D, _B).transpose(2, 0, 1).reshape(_B, _G * _D)


# speedup vs baseline: 176.4877x; 1.2592x over previous
import jax
import jax.numpy as jnp
from jax import lax
from jax.experimental import pallas as pl
from jax.experimental.pallas import tpu as pltpu
from jax.experimental.pallas import tpu_sc as plsc

_A = 8
_G = 256
_D = 16
_B = 1024
_NW = 32
_GPW = _G // _NW
_QB = _B // 4
_NCH = _QB // 16
_TSZ = 1 << _A
_GSZ = _TSZ * _D

_NET = [(0, 1), (2, 3), (4, 5), (6, 7),
        (0, 2), (1, 3), (4, 6), (5, 7),
        (1, 2), (5, 6), (0, 4), (3, 7),
        (1, 5), (2, 6),
        (1, 4), (3, 6),
        (2, 4), (3, 5),
        (3, 4)]


def _sc_body(xt_hbm, p_hbm, out_hbm, x_v, t_v, o_v):
    wid = lax.axis_index("c") * 16 + lax.axis_index("s")

    pconst = [jnp.full((16,), (1 << k) * _D, jnp.int32) for k in range(_A)]

    def table_body(gi, carry):
        pltpu.sync_copy(p_hbm.at[wid * _GPW + gi],
                        t_v.at[pl.ds(gi * _GSZ, _GSZ)])
        return carry

    lax.fori_loop(0, _GPW, table_body, 0)

    def quarter_body(q, carry):
        def xload_body(gi, c2):
            pltpu.sync_copy(
                xt_hbm.at[wid * _GPW + gi, :, pl.ds(q * _QB, _QB)],
                x_v.at[gi])
            return c2

        lax.fori_loop(0, _GPW, xload_body, 0)

        def group_body(gi, c2):
            mask_init = jnp.full((16,), 255 * _D, jnp.int32) + jnp.full(
                (16,), gi * _GSZ, jnp.int32)

            def chunk_body(c, c3):
                base = c * 16
                v = [x_v[gi, k, pl.ds(base, 16)] for k in range(_A)]
                p = list(pconst)

                for a, b in _NET:
                    le = v[a] <= v[b]
                    va = jnp.minimum(v[a], v[b])
                    vb = jnp.maximum(v[a], v[b])
                    pa = jnp.where(le, p[a], p[b])
                    pb = jnp.where(le, p[b], p[a])
                    v[a], v[b], p[a], p[b] = va, vb, pa, pb

                mask = mask_init
                masks = [None] * _A
                coefs = [None] * _A
                for r in range(_A):
                    masks[r] = mask
                    coefs[r] = v[r] if r == 0 else v[r] - v[r - 1]
                    if r < _A - 1:
                        mask = mask - p[r]

                for pp in range(16):
                    acc = None
                    for r in range(_A):
                        addr = masks[r][pp]
                        row = t_v[pl.ds(addr, 16)]
                        cf = coefs[r][pp]
                        acc = row * cf if r == 0 else acc + row * cf
                    o_v[base + pp, pl.ds(gi * _D, _D)] = acc
                return c3

            lax.fori_loop(0, _NCH, chunk_body, 0)
            return c2

        lax.fori_loop(0, _GPW, group_body, 0)
        pltpu.sync_copy(
            o_v, out_hbm.at[pl.ds(q * _QB, _QB), pl.ds(wid * _GPW * _D, _GPW * _D)])
        return carry

    lax.fori_loop(0, 4, quarter_body, 0)


_sc_call = pl.kernel(
    _sc_body,
    out_type=jax.ShapeDtypeStruct((_B, _G * _D), jnp.float32),
    mesh=plsc.VectorSubcoreMesh(core_axis_name="c", subcore_axis_name="s"),
    compiler_params=pltpu.CompilerParams(needs_layout_passes=False),
    scratch_types=[
        pltpu.VMEM((_GPW, _A, _QB), jnp.float32),
        pltpu.VMEM((_GPW * _GSZ,), jnp.float32),
        pltpu.VMEM((_QB, _GPW * _D), jnp.float32),
    ],
)


def kernel(X, params):
    xt = X.reshape(_B, _G, _A).transpose(1, 2, 0)
    return _sc_call(xt, params.reshape(_G, _GSZ))


# speedup vs baseline: 191.1607x; 1.0831x over previous
import jax
import jax.numpy as jnp
from jax import lax
from jax.experimental import pallas as pl
from jax.experimental.pallas import tpu as pltpu
from jax.experimental.pallas import tpu_sc as plsc

_A = 8
_G = 256
_D = 16
_B = 1024
_NW = 32
_GPW = _G // _NW
_QB = _B // 4
_NCH = _QB // 16
_TSZ = 1 << _A
_GSZ = _TSZ * _D

_NET = [(0, 1), (2, 3), (4, 5), (6, 7),
        (0, 2), (1, 3), (4, 6), (5, 7),
        (1, 2), (5, 6), (0, 4), (3, 7),
        (1, 5), (2, 6),
        (1, 4), (3, 6),
        (2, 4), (3, 5),
        (3, 4)]


def _sc_body(xt_hbm, p_hbm, out_hbm, x_v, t_v, o_v):
    wid = lax.axis_index("c") * 16 + lax.axis_index("s")

    pconst = [jnp.full((16,), (1 << k) * _D, jnp.int32) for k in range(_A)]
    iota = lax.iota(jnp.int32, 16)

    def table_body(gi, carry):
        pltpu.sync_copy(p_hbm.at[wid * _GPW + gi],
                        t_v.at[pl.ds(gi * _GSZ, _GSZ)])
        return carry

    lax.fori_loop(0, _GPW, table_body, 0)

    def quarter_body(q, carry):
        def xload_body(gi, c2):
            pltpu.sync_copy(
                xt_hbm.at[wid * _GPW + gi, :, pl.ds(q * _QB, _QB)],
                x_v.at[gi])
            return c2

        lax.fori_loop(0, _GPW, xload_body, 0)

        def group_body(gi, c2):
            mask_init = jnp.full((16,), 255 * _D, jnp.int32) + jnp.full(
                (16,), gi * _GSZ, jnp.int32)
            row0 = plsc.load_gather(t_v, [mask_init + iota])

            def sort_phase(c):
                v = [x_v[gi, k, pl.ds(c * 16, 16)] for k in range(_A)]
                p = list(pconst)

                for a, b in _NET:
                    le = v[a] <= v[b]
                    va = jnp.minimum(v[a], v[b])
                    vb = jnp.maximum(v[a], v[b])
                    pa = jnp.where(le, p[a], p[b])
                    pb = jnp.where(le, p[b], p[a])
                    v[a], v[b], p[a], p[b] = va, vb, pa, pb

                mask = mask_init
                masks = [None] * _A
                coefs = [None] * _A
                for r in range(_A):
                    masks[r] = mask
                    coefs[r] = v[r] if r == 0 else v[r] - v[r - 1]
                    if r < _A - 1:
                        mask = mask - p[r]
                return tuple(masks), tuple(coefs)

            def gather_phase(c, masks, coefs):
                base = c * 16
                for pp in range(16):
                    terms = [row0 * coefs[0][pp]]
                    for r in range(1, _A):
                        idx = jnp.full((16,), masks[r][pp], jnp.int32) + iota
                        row = plsc.load_gather(t_v, [idx])
                        terms.append(row * coefs[r][pp])
                    while len(terms) > 1:
                        terms = [terms[i] + terms[i + 1]
                                 for i in range(0, len(terms), 2)]
                    o_v[base + pp, pl.ds(gi * _D, _D)] = terms[0]

            def chunk_body(c, carry):
                masks, coefs = carry
                gather_phase(c, masks, coefs)
                return sort_phase(c + 1)

            last = lax.fori_loop(0, _NCH - 1, chunk_body, sort_phase(0))
            gather_phase(_NCH - 1, *last)
            return c2

        lax.fori_loop(0, _GPW, group_body, 0)
        pltpu.sync_copy(
            o_v, out_hbm.at[pl.ds(q * _QB, _QB), pl.ds(wid * _GPW * _D, _GPW * _D)])
        return carry

    lax.fori_loop(0, 4, quarter_body, 0)


_sc_call = pl.kernel(
    _sc_body,
    out_type=jax.ShapeDtypeStruct((_B, _G * _D), jnp.float32),
    mesh=plsc.VectorSubcoreMesh(core_axis_name="c", subcore_axis_name="s"),
    compiler_params=pltpu.CompilerParams(needs_layout_passes=False),
    scratch_types=[
        pltpu.VMEM((_GPW, _A, _QB), jnp.float32),
        pltpu.VMEM((_GPW * _GSZ,), jnp.float32),
        pltpu.VMEM((_QB, _GPW * _D), jnp.float32),
    ],
)


def kernel(X, params):
    xt = X.reshape(_B, _G, _A).transpose(1, 2, 0)
    return _sc_call(xt, params.reshape(_G, _GSZ))


# speedup vs baseline: 215.4906x; 1.1273x over previous
import jax
import jax.numpy as jnp
from jax import lax
from jax.experimental import pallas as pl
from jax.experimental.pallas import tpu as pltpu
from jax.experimental.pallas import tpu_sc as plsc

_A = 8
_G = 256
_D = 16
_B = 1024
_NW = 32
_GPW = _G // _NW
_QB = _B // 4
_NCH = _QB // 16
_TSZ = 1 << _A
_GSZ = _TSZ * _D

_NET = [(0, 1), (2, 3), (4, 5), (6, 7),
        (0, 2), (1, 3), (4, 6), (5, 7),
        (1, 2), (5, 6), (0, 4), (3, 7),
        (1, 5), (2, 6),
        (1, 4), (3, 6),
        (2, 4), (3, 5),
        (3, 4)]


def _sc_body(xt_hbm, p_hbm, out_hbm, x_v, t_v, o_v, t_sem):
    wid = lax.axis_index("c") * 16 + lax.axis_index("s")

    pconst = [jnp.full((16,), (1 << k) * _D, jnp.int32) for k in range(_A)]
    iota = lax.iota(jnp.int32, 16)

    tcps = [pltpu.async_copy(p_hbm.at[wid * _GPW + gi2],
                             t_v.at[pl.ds(gi2 * _GSZ, _GSZ)], t_sem)
            for gi2 in range(_GPW)]
    for cp in tcps:
        cp.wait()

    def quarter_body(q, carry):
        pltpu.sync_copy(
            xt_hbm.at[pl.ds(wid * _GPW, _GPW), :, pl.ds(q * _QB, _QB)],
            x_v)

        def group_body(gi, c2):
            mask_init = jnp.full((16,), 255 * _D, jnp.int32) + jnp.full(
                (16,), gi * _GSZ, jnp.int32)
            row0 = plsc.load_gather(t_v, [mask_init + iota])

            def sort_phase(c):
                v = [x_v[gi, k, pl.ds(c * 16, 16)] for k in range(_A)]
                p = list(pconst)

                for a, b in _NET:
                    le = v[a] <= v[b]
                    va = jnp.minimum(v[a], v[b])
                    vb = jnp.maximum(v[a], v[b])
                    pa = jnp.where(le, p[a], p[b])
                    pb = jnp.where(le, p[b], p[a])
                    v[a], v[b], p[a], p[b] = va, vb, pa, pb

                mask = mask_init
                masks = [None] * _A
                coefs = [None] * _A
                for r in range(_A):
                    masks[r] = mask
                    coefs[r] = v[r] if r == 0 else v[r] - v[r - 1]
                    if r < _A - 1:
                        mask = mask - p[r]
                return tuple(masks), tuple(coefs)

            def gather_phase(c, masks, coefs):
                base = c * 16
                for pp in range(16):
                    terms = [row0 * coefs[0][pp]]
                    for r in range(1, _A):
                        idx = jnp.full((16,), masks[r][pp], jnp.int32) + iota
                        row = plsc.load_gather(t_v, [idx])
                        terms.append(row * coefs[r][pp])
                    while len(terms) > 1:
                        terms = [terms[i] + terms[i + 1]
                                 for i in range(0, len(terms), 2)]
                    o_v[base + pp, pl.ds(gi * _D, _D)] = terms[0]

            def chunk_body(c, carry):
                masks, coefs = carry
                gather_phase(c, masks, coefs)
                return sort_phase(c + 1)

            last = lax.fori_loop(0, _NCH - 1, chunk_body, sort_phase(0))
            gather_phase(_NCH - 1, *last)
            return c2

        lax.fori_loop(0, _GPW, group_body, 0)
        pltpu.sync_copy(
            o_v, out_hbm.at[pl.ds(q * _QB, _QB), pl.ds(wid * _GPW * _D, _GPW * _D)])
        return carry

    lax.fori_loop(0, 4, quarter_body, 0)


_sc_call = pl.kernel(
    _sc_body,
    out_type=jax.ShapeDtypeStruct((_B, _G * _D), jnp.float32),
    mesh=plsc.VectorSubcoreMesh(core_axis_name="c", subcore_axis_name="s"),
    compiler_params=pltpu.CompilerParams(needs_layout_passes=False),
    scratch_types=[
        pltpu.VMEM((_GPW, _A, _QB), jnp.float32),
        pltpu.VMEM((_GPW * _GSZ,), jnp.float32),
        pltpu.VMEM((_QB, _GPW * _D), jnp.float32),
        pltpu.SemaphoreType.DMA,
    ],
)


def kernel(X, params):
    xt = X.reshape(_B, _G, _A).transpose(1, 2, 0)
    return _sc_call(xt, params.reshape(_G, _GSZ))


# speedup vs baseline: 278.9049x; 1.2943x over previous
import jax
import jax.numpy as jnp
from jax import lax
from jax.experimental import pallas as pl
from jax.experimental.pallas import tpu as pltpu
from jax.experimental.pallas import tpu_sc as plsc

_A = 8
_G = 256
_D = 16
_B = 1024
_NW = 32
_GPW = _G // _NW
_QB = _B // 4
_NCH = _QB // 16
_TSZ = 1 << _A
_GSZ = _TSZ * _D

_NET = [(0, 1), (2, 3), (4, 5), (6, 7),
        (0, 2), (1, 3), (4, 6), (5, 7),
        (1, 2), (5, 6), (0, 4), (3, 7),
        (1, 5), (2, 6),
        (1, 4), (3, 6),
        (2, 4), (3, 5),
        (3, 4)]


def _sc_body(xt_hbm, p_hbm, out_hbm, x_v, t_v, o_v, t_sem):
    wid = lax.axis_index("c") * 16 + lax.axis_index("s")

    pconst = [jnp.full((16,), (1 << k) * _D, jnp.int32) for k in range(_A)]
    iota = lax.iota(jnp.int32, 16)

    tcps = [pltpu.async_copy(p_hbm.at[wid * _GPW + gi2],
                             t_v.at[pl.ds(gi2 * _GSZ, _GSZ)], t_sem)
            for gi2 in range(_GPW)]
    for cp in tcps:
        cp.wait()

    def quarter_body(q, carry):
        pltpu.sync_copy(
            xt_hbm.at[pl.ds(wid * _GPW, _GPW), :, pl.ds(q * _QB, _QB)],
            x_v)

        def group_body(gi, c2):
            mask_init = jnp.full((16,), 255 * _D, jnp.int32) + jnp.full(
                (16,), gi * _GSZ, jnp.int32)
            row0 = plsc.load_gather(t_v, [mask_init + iota])

            def sort_phase(c):
                v = [x_v[gi, k, pl.ds(c * 16, 16)] for k in range(_A)]
                p = list(pconst)

                for a, b in _NET:
                    le = v[a] <= v[b]
                    va = jnp.minimum(v[a], v[b])
                    vb = jnp.maximum(v[a], v[b])
                    pa = jnp.where(le, p[a], p[b])
                    pb = jnp.where(le, p[b], p[a])
                    v[a], v[b], p[a], p[b] = va, vb, pa, pb

                mask = mask_init
                masks = [None] * _A
                coefs = [None] * _A
                for r in range(_A):
                    masks[r] = mask
                    coefs[r] = v[r] if r == 0 else v[r] - v[r - 1]
                    if r < _A - 1:
                        mask = mask - p[r]
                return tuple(masks), tuple(coefs)

            def gather_phase(c, masks, coefs):
                base = c * 16
                for p0 in range(0, 16, 4):
                    rows = {}
                    for pp in range(p0, p0 + 4):
                        for r in range(1, _A):
                            idx = jnp.full((16,), masks[r][pp],
                                           jnp.int32) + iota
                            rows[pp, r] = plsc.load_gather(t_v, [idx])
                    for pp in range(p0, p0 + 4):
                        terms = [row0 * coefs[0][pp]]
                        terms += [rows[pp, r] * coefs[r][pp]
                                  for r in range(1, _A)]
                        while len(terms) > 1:
                            terms = [terms[i] + terms[i + 1]
                                     for i in range(0, len(terms), 2)]
                        o_v[base + pp, pl.ds(gi * _D, _D)] = terms[0]

            def chunk_body(c, carry):
                masks, coefs = sort_phase(c)
                gather_phase(c, masks, coefs)
                return carry

            lax.fori_loop(0, _NCH, chunk_body, 0)
            return c2

        lax.fori_loop(0, _GPW, group_body, 0)
        pltpu.sync_copy(
            o_v, out_hbm.at[pl.ds(q * _QB, _QB), pl.ds(wid * _GPW * _D, _GPW * _D)])
        return carry

    lax.fori_loop(0, 4, quarter_body, 0)


_sc_call = pl.kernel(
    _sc_body,
    out_type=jax.ShapeDtypeStruct((_B, _G * _D), jnp.float32),
    mesh=plsc.VectorSubcoreMesh(core_axis_name="c", subcore_axis_name="s"),
    compiler_params=pltpu.CompilerParams(needs_layout_passes=False),
    scratch_types=[
        pltpu.VMEM((_GPW, _A, _QB), jnp.float32),
        pltpu.VMEM((_GPW * _GSZ,), jnp.float32),
        pltpu.VMEM((_QB, _GPW * _D), jnp.float32),
        pltpu.SemaphoreType.DMA,
    ],
)


def kernel(X, params):
    xt = X.reshape(_B, _G, _A).transpose(1, 2, 0)
    return _sc_call(xt, params.reshape(_G, _GSZ))


# speedup vs baseline: 280.3553x; 1.0052x over previous
import jax
import jax.numpy as jnp
from jax import lax
from jax.experimental import pallas as pl
from jax.experimental.pallas import tpu as pltpu
from jax.experimental.pallas import tpu_sc as plsc

_A = 8
_G = 256
_D = 16
_B = 1024
_NW = 32
_GPW = _G // _NW
_NP = 8
_QB = _B // _NP
_NCH = _QB // 16
_TSZ = 1 << _A
_GSZ = _TSZ * _D

_NET = [(0, 1), (2, 3), (4, 5), (6, 7),
        (0, 2), (1, 3), (4, 6), (5, 7),
        (1, 2), (5, 6), (0, 4), (3, 7),
        (1, 5), (2, 6),
        (1, 4), (3, 6),
        (2, 4), (3, 5),
        (3, 4)]


def _sc_body(xt_hbm, p_hbm, out_hbm, x_v, t_v, o_v, t_sem, o_sem):
    wid = lax.axis_index("c") * 16 + lax.axis_index("s")

    pconst = [jnp.full((16,), (1 << k) * _D, jnp.int32) for k in range(_A)]
    iota = lax.iota(jnp.int32, 16)

    tcps = [pltpu.async_copy(p_hbm.at[wid * _GPW + gi2],
                             t_v.at[pl.ds(gi2 * _GSZ, _GSZ)], t_sem)
            for gi2 in range(_GPW)]
    for cp in tcps:
        cp.wait()

    def out_win(q):
        return out_hbm.at[pl.ds(q * _QB, _QB),
                          pl.ds(wid * _GPW * _D, _GPW * _D)]

    def quarter_body(q, carry):
        par = q % 2
        pltpu.sync_copy(
            xt_hbm.at[pl.ds(wid * _GPW, _GPW), :, pl.ds(q * _QB, _QB)],
            x_v)

        @pl.when(q >= 2)
        def _():
            pltpu.make_async_copy(o_v.at[par], out_win(q - 2), o_sem).wait()

        def group_body(gi, c2):
            mask_init = jnp.full((16,), 255 * _D, jnp.int32) + jnp.full(
                (16,), gi * _GSZ, jnp.int32)
            row0 = plsc.load_gather(t_v, [mask_init + iota])

            def sort_phase(c):
                v = [x_v[gi, k, pl.ds(c * 16, 16)] for k in range(_A)]
                p = list(pconst)

                for a, b in _NET:
                    le = v[a] <= v[b]
                    va = jnp.minimum(v[a], v[b])
                    vb = jnp.maximum(v[a], v[b])
                    pa = jnp.where(le, p[a], p[b])
                    pb = jnp.where(le, p[b], p[a])
                    v[a], v[b], p[a], p[b] = va, vb, pa, pb

                mask = mask_init
                masks = [None] * _A
                coefs = [None] * _A
                for r in range(_A):
                    masks[r] = mask
                    coefs[r] = v[r] if r == 0 else v[r] - v[r - 1]
                    if r < _A - 1:
                        mask = mask - p[r]
                return tuple(masks), tuple(coefs)

            def gather_phase(c, masks, coefs):
                base = c * 16
                for p0 in range(0, 16, 4):
                    rows = {}
                    for pp in range(p0, p0 + 4):
                        for r in range(1, _A):
                            idx = jnp.full((16,), masks[r][pp],
                                           jnp.int32) + iota
                            rows[pp, r] = plsc.load_gather(t_v, [idx])
                    for pp in range(p0, p0 + 4):
                        terms = [row0 * coefs[0][pp]]
                        terms += [rows[pp, r] * coefs[r][pp]
                                  for r in range(1, _A)]
                        while len(terms) > 1:
                            terms = [terms[i] + terms[i + 1]
                                     for i in range(0, len(terms), 2)]
                        o_v[par, base + pp, pl.ds(gi * _D, _D)] = terms[0]

            def chunk_body(c, carry):
                masks, coefs = sort_phase(c)
                gather_phase(c, masks, coefs)
                return carry

            lax.fori_loop(0, _NCH, chunk_body, 0)
            return c2

        lax.fori_loop(0, _GPW, group_body, 0)
        pltpu.async_copy(o_v.at[par], out_win(q), o_sem)
        return carry

    lax.fori_loop(0, _NP, quarter_body, 0)
    for j in (_NP - 2, _NP - 1):
        pltpu.make_async_copy(o_v.at[j % 2], out_win(j), o_sem).wait()


_sc_call = pl.kernel(
    _sc_body,
    out_type=jax.ShapeDtypeStruct((_B, _G * _D), jnp.float32),
    mesh=plsc.VectorSubcoreMesh(core_axis_name="c", subcore_axis_name="s"),
    compiler_params=pltpu.CompilerParams(needs_layout_passes=False),
    scratch_types=[
        pltpu.VMEM((_GPW, _A, _QB), jnp.float32),
        pltpu.VMEM((_GPW * _GSZ,), jnp.float32),
        pltpu.VMEM((2, _QB, _GPW * _D), jnp.float32),
        pltpu.SemaphoreType.DMA,
        pltpu.SemaphoreType.DMA,
    ],
)


def kernel(X, params):
    xt = X.reshape(_B, _G, _A).transpose(1, 2, 0)
    return _sc_call(xt, params.reshape(_G, _GSZ))
